# baseline (speedup 1.0000x reference)
import functools

import jax
import jax.numpy as jnp
import numpy as np
from jax.experimental import pallas as pl
from jax.experimental.pallas import tpu as pltpu

V = 10000
E = 320000
B = 8
C = 128
K = 3
OUT = 128

M_TILE = 1000


def _linear_body(x0_ref, x1_ref, s2_ref, w_ref, b_ref, o_ref):
    w0 = w_ref[0]
    w1 = w_ref[1]
    w2 = w_ref[2]
    acc = jnp.dot(x0_ref[...], w0 - w2, preferred_element_type=jnp.float32)
    acc = acc + jnp.dot(x1_ref[...], w1, preferred_element_type=jnp.float32)
    acc = acc + jnp.dot(s2_ref[...], 2.0 * w2, preferred_element_type=jnp.float32)
    o_ref[...] = acc + b_ref[...]


def _linear(x0, x1, s2, w, b):
    m = B * V
    grid = (m // M_TILE,)
    return pl.pallas_call(
        _linear_body,
        grid=grid,
        in_specs=[
            pl.BlockSpec((M_TILE, C), lambda i: (i, 0)),
            pl.BlockSpec((M_TILE, C), lambda i: (i, 0)),
            pl.BlockSpec((M_TILE, C), lambda i: (i, 0)),
            pl.BlockSpec((K, C, OUT), lambda i: (0, 0, 0)),
            pl.BlockSpec((1, OUT), lambda i: (0, 0)),
        ],
        out_specs=pl.BlockSpec((M_TILE, OUT), lambda i: (i, 0)),
        out_shape=jax.ShapeDtypeStruct((m, OUT), jnp.float32),
    )(x0, x1, s2, w, b)


def _spmm_jnp(rows, cols, vals, x):
    xb = x.reshape(B, V, C)
    gathered = xb[:, cols, :] * vals[None, :, None]
    y = jax.ops.segment_sum(gathered.reshape(B * E, C),
                            (jnp.arange(B)[:, None] * V + rows[None, :]).reshape(-1),
                            num_segments=B * V)
    return y


def kernel(input, lap_rows, lap_cols, lap_vals, W, b):
    x0 = jnp.transpose(input, (0, 2, 1)).reshape(B * V, C)
    y1 = _spmm_jnp(lap_rows, lap_cols, lap_vals, x0)
    s2 = _spmm_jnp(lap_rows, lap_cols, lap_vals, y1)
    wp = jnp.transpose(W.reshape(C, K, OUT), (1, 0, 2))
    out = _linear(x0, y1, s2, wp, b.reshape(1, OUT))
    return jnp.transpose(out.reshape(B, V, OUT), (0, 2, 1))


# speedup vs baseline: 2.7122x; 2.7122x over previous
import functools

import jax
import jax.numpy as jnp
from jax import lax
from jax.experimental import pallas as pl
from jax.experimental.pallas import tpu as pltpu
from jax.experimental.pallas import tpu_sc as plsc

V = 10000
E = 320000
B = 8
C = 128
K = 3
OUT = 128

NS = 16
NC = 2
CH = 128
NCH = 157
EPAD = NS * NCH * CH
RPS = 624
ZB = 16
TAIL = V - NS * RPS
BPC = B // NC
NEB = 3

M_TILE = 1000


def _spmm_sc(edges, evals, x_flat):
    mesh = plsc.VectorSubcoreMesh(core_axis_name="c", subcore_axis_name="s")

    @functools.partial(
        pl.kernel,
        mesh=mesh,
        out_type=jax.ShapeDtypeStruct((B, V, C), jnp.float32),
        scratch_types=[
            pltpu.VMEM((NEB, 2, CH), jnp.int32),
            pltpu.VMEM((NEB, CH), jnp.float32),
            pltpu.VMEM((2, CH), jnp.int32),
            pltpu.VMEM((2, CH, C), jnp.float32),
            pltpu.VMEM((ZB, C), jnp.float32),
            pltpu.VMEM_SHARED((V, C), jnp.float32),
            pltpu.SemaphoreType.DMA((NEB,)),
            pltpu.SemaphoreType.DMA((2,)),
        ],
    )
    def k(edges_h, evals_h, x_h, y_h, ebuf, vbuf, idxb, gbuf, zbuf, acc,
          esems, gsems):
        cid = lax.axis_index("c")
        sid = lax.axis_index("s")

        zero16 = jnp.zeros((16,), jnp.float32)
        for r in range(ZB):
            for j in range(C // 16):
                zbuf[r, pl.ds(16 * j, 16)] = zero16

        def fire_edges(t, es):
            pltpu.make_async_copy(
                edges_h.at[sid, t], ebuf.at[es], esems.at[es]).start()
            pltpu.make_async_copy(
                evals_h.at[sid, t], vbuf.at[es], esems.at[es]).start()

        def wait_edges(t, es):
            pltpu.make_async_copy(
                edges_h.at[sid, t], ebuf.at[es], esems.at[es]).wait()
            pltpu.make_async_copy(
                evals_h.at[sid, t], vbuf.at[es], esems.at[es]).wait()

        for bi in range(BPC):
            b = cid * BPC + bi
            boff = jnp.full((16,), b * V, jnp.int32)

            for i in range(RPS // ZB):
                pltpu.sync_copy(zbuf, acc.at[pl.ds(sid * RPS + i * ZB, ZB)])

            @pl.when(sid == NS - 1)
            def _():
                pltpu.sync_copy(zbuf.at[pl.ds(0, TAIL)],
                                acc.at[pl.ds(NS * RPS, TAIL)])

            plsc.subcore_barrier()

            def stage_and_fire_gather(t, es, slot):
                for g in range(CH // 16):
                    sl = pl.ds(16 * g, 16)
                    idxb[slot, sl] = ebuf[es, 1, sl] + boff
                pltpu.make_async_copy(
                    x_h.at[idxb.at[slot]], gbuf.at[slot], gsems.at[slot]
                ).start()

            fire_edges(0, 0)
            fire_edges(1, 1)
            wait_edges(0, 0)
            stage_and_fire_gather(0, 0, 0)

            def chunk(t, carry):
                slot = lax.rem(t, 2)
                es = lax.rem(t, NEB)

                @pl.when(t + 2 < NCH)
                def _():
                    fire_edges(t + 2, lax.rem(t + 2, NEB))

                @pl.when(t + 1 < NCH)
                def _():
                    wait_edges(t + 1, lax.rem(t + 1, NEB))
                    stage_and_fire_gather(t + 1, lax.rem(t + 1, NEB), 1 - slot)

                pltpu.make_async_copy(
                    x_h.at[idxb.at[slot]], gbuf.at[slot], gsems.at[slot]
                ).wait()

                def grp(g, gcarry):
                    vv = vbuf[es, pl.ds(16 * g, 16)]
                    e0 = 16 * g
                    for l in range(16):
                        v = vv[l]
                        for j in range(C // 16):
                            sl = pl.ds(16 * j, 16)
                            gbuf[slot, e0 + l, sl] = gbuf[slot, e0 + l, sl] * v
                    return gcarry

                lax.fori_loop(0, CH // 16, grp, 0)
                pltpu.sync_copy(gbuf.at[slot], acc.at[ebuf.at[es, 0]],
                                add=True)
                return carry

            lax.fori_loop(0, NCH, chunk, 0)
            plsc.subcore_barrier()

            for i in range(RPS // ZB):
                r0 = sid * RPS + i * ZB
                pltpu.sync_copy(acc.at[pl.ds(r0, ZB)],
                                y_h.at[b, pl.ds(r0, ZB)])

            @pl.when(sid == NS - 1)
            def _():
                r0 = NS * RPS
                pltpu.sync_copy(acc.at[pl.ds(r0, TAIL)],
                                y_h.at[b, pl.ds(r0, TAIL)])

    return k(edges, evals, x_flat)


def _linear_body(x0_ref, x1_ref, s2_ref, w_ref, b_ref, o_ref):
    w0 = w_ref[0]
    w1 = w_ref[1]
    w2 = w_ref[2]
    acc = jnp.dot(x0_ref[...], w0 - w2, preferred_element_type=jnp.float32)
    acc = acc + jnp.dot(x1_ref[...], w1, preferred_element_type=jnp.float32)
    acc = acc + jnp.dot(s2_ref[...], 2.0 * w2, preferred_element_type=jnp.float32)
    o_ref[...] = acc + b_ref[...]


def _linear(x0, x1, s2, w, b):
    m = B * V
    grid = (m // M_TILE,)
    return pl.pallas_call(
        _linear_body,
        grid=grid,
        in_specs=[
            pl.BlockSpec((M_TILE, C), lambda i: (i, 0)),
            pl.BlockSpec((M_TILE, C), lambda i: (i, 0)),
            pl.BlockSpec((M_TILE, C), lambda i: (i, 0)),
            pl.BlockSpec((K, C, OUT), lambda i: (0, 0, 0)),
            pl.BlockSpec((1, OUT), lambda i: (0, 0)),
        ],
        out_specs=pl.BlockSpec((M_TILE, OUT), lambda i: (i, 0)),
        out_shape=jax.ShapeDtypeStruct((m, OUT), jnp.float32),
    )(x0, x1, s2, w, b)


def kernel(input, lap_rows, lap_cols, lap_vals, W, b):
    x0 = jnp.transpose(input, (0, 2, 1)).reshape(B * V, C)

    pad = EPAD - E
    r = jnp.concatenate([lap_rows, jnp.zeros((pad,), jnp.int32)])
    c = jnp.concatenate([lap_cols, jnp.zeros((pad,), jnp.int32)])
    v = jnp.concatenate([lap_vals, jnp.zeros((pad,), jnp.float32)])
    edges = jnp.stack(
        [r.reshape(NS, NCH, CH), c.reshape(NS, NCH, CH)], axis=2)
    evals = v.reshape(NS, NCH, CH)

    y1 = _spmm_sc(edges, evals, x0)
    s2 = _spmm_sc(edges, evals, y1.reshape(B * V, C))

    wp = jnp.transpose(W.reshape(C, K, OUT), (1, 0, 2))
    out = _linear(x0, y1.reshape(B * V, C), s2.reshape(B * V, C),
                  wp, b.reshape(1, OUT))
    return jnp.transpose(out.reshape(B, V, OUT), (0, 2, 1))


# speedup vs baseline: 2.9483x; 1.0870x over previous
import functools

import jax
import jax.numpy as jnp
from jax import lax
from jax.experimental import pallas as pl
from jax.experimental.pallas import tpu as pltpu
from jax.experimental.pallas import tpu_sc as plsc

V = 10000
E = 320000
B = 8
C = 128
K = 3
OUT = 128

NS = 16
NC = 2
CH = 112
NCH = 180
EPAD = NS * NCH * CH
RPS = 624
ZB = 16
TAIL = V - NS * RPS
BPC = B // NC
NEB = 4
NGB = 3

M_TILE = 1000


def _spmm_sc(edges, evals, x_flat):
    mesh = plsc.VectorSubcoreMesh(core_axis_name="c", subcore_axis_name="s")

    @functools.partial(
        pl.kernel,
        mesh=mesh,
        out_type=jax.ShapeDtypeStruct((B, V, C), jnp.float32),
        scratch_types=[
            pltpu.VMEM((NEB, 2, CH), jnp.int32),
            pltpu.VMEM((NEB, CH), jnp.float32),
            pltpu.VMEM((2, CH), jnp.int32),
            pltpu.VMEM((NGB, CH, C), jnp.float32),
            pltpu.VMEM((ZB, C), jnp.float32),
            pltpu.VMEM_SHARED((V, C), jnp.float32),
            pltpu.SemaphoreType.DMA((NEB,)),
            pltpu.SemaphoreType.DMA((NGB,)),
            pltpu.SemaphoreType.DMA((2,)),
        ],
    )
    def k(edges_h, evals_h, x_h, y_h, ebuf, vbuf, idxb, gbuf, zbuf, acc,
          esems, gsems, ssems):
        cid = lax.axis_index("c")
        sid = lax.axis_index("s")

        zero16 = jnp.zeros((16,), jnp.float32)
        for r in range(ZB):
            for j in range(C // 16):
                zbuf[r, pl.ds(16 * j, 16)] = zero16

        def fire_edges(t, es):
            pltpu.make_async_copy(
                edges_h.at[sid, t], ebuf.at[es], esems.at[es]).start()
            pltpu.make_async_copy(
                evals_h.at[sid, t], vbuf.at[es], esems.at[es]).start()

        def wait_edges(t, es):
            pltpu.make_async_copy(
                edges_h.at[sid, t], ebuf.at[es], esems.at[es]).wait()
            pltpu.make_async_copy(
                evals_h.at[sid, t], vbuf.at[es], esems.at[es]).wait()

        for bi in range(BPC):
            b = cid * BPC + bi
            boff = jnp.full((16,), b * V, jnp.int32)

            for i in range(RPS // ZB):
                pltpu.sync_copy(zbuf, acc.at[pl.ds(sid * RPS + i * ZB, ZB)])

            @pl.when(sid == NS - 1)
            def _():
                pltpu.sync_copy(zbuf.at[pl.ds(0, TAIL)],
                                acc.at[pl.ds(NS * RPS, TAIL)])

            plsc.subcore_barrier()

            def stage_and_fire_gather(t, es, islot, gs):
                for g in range(CH // 16):
                    sl = pl.ds(16 * g, 16)
                    idxb[islot, sl] = ebuf[es, 1, sl] + boff
                pltpu.make_async_copy(
                    x_h.at[idxb.at[islot]], gbuf.at[gs], gsems.at[gs]
                ).start()

            def scatter_desc(gs, es, sslot):
                return pltpu.make_async_copy(
                    gbuf.at[gs], acc.at[ebuf.at[es, 0]], ssems.at[sslot])

            fire_edges(0, 0)
            fire_edges(1, 1)
            wait_edges(0, 0)
            stage_and_fire_gather(0, 0, 0, 0)

            def chunk(t, carry):
                gs = lax.rem(t, NGB)
                es = lax.rem(t, NEB)

                @pl.when(t >= 2)
                def _():
                    scatter_desc(lax.rem(t - 2, NGB), lax.rem(t - 2, NEB),
                                 lax.rem(t - 2, 2)).wait()

                @pl.when(t + 2 < NCH)
                def _():
                    fire_edges(t + 2, lax.rem(t + 2, NEB))

                @pl.when(t + 1 < NCH)
                def _():
                    wait_edges(t + 1, lax.rem(t + 1, NEB))
                    stage_and_fire_gather(t + 1, lax.rem(t + 1, NEB),
                                          lax.rem(t + 1, 2),
                                          lax.rem(t + 1, NGB))

                pltpu.make_async_copy(
                    x_h.at[idxb.at[lax.rem(t, 2)]], gbuf.at[gs], gsems.at[gs]
                ).wait()

                def grp(g, gcarry):
                    vv = vbuf[es, pl.ds(16 * g, 16)]
                    e0 = 16 * g
                    for l in range(16):
                        v = vv[l]
                        for j in range(C // 16):
                            sl = pl.ds(16 * j, 16)
                            gbuf[gs, e0 + l, sl] = gbuf[gs, e0 + l, sl] * v
                    return gcarry

                lax.fori_loop(0, CH // 16, grp, 0)
                scatter_desc(gs, es, lax.rem(t, 2)).start(add=True)
                return carry

            lax.fori_loop(0, NCH, chunk, 0)
            scatter_desc(lax.rem(NCH - 2, NGB), lax.rem(NCH - 2, NEB),
                         lax.rem(NCH - 2, 2)).wait()
            scatter_desc(lax.rem(NCH - 1, NGB), lax.rem(NCH - 1, NEB),
                         lax.rem(NCH - 1, 2)).wait()
            plsc.subcore_barrier()

            for i in range(RPS // ZB):
                r0 = sid * RPS + i * ZB
                pltpu.sync_copy(acc.at[pl.ds(r0, ZB)],
                                y_h.at[b, pl.ds(r0, ZB)])

            @pl.when(sid == NS - 1)
            def _():
                r0 = NS * RPS
                pltpu.sync_copy(acc.at[pl.ds(r0, TAIL)],
                                y_h.at[b, pl.ds(r0, TAIL)])

    return k(edges, evals, x_flat)


def _linear_body(x0_ref, x1_ref, s2_ref, w_ref, b_ref, o_ref):
    w0 = w_ref[0]
    w1 = w_ref[1]
    w2 = w_ref[2]
    acc = jnp.dot(x0_ref[...], w0 - w2, preferred_element_type=jnp.float32)
    acc = acc + jnp.dot(x1_ref[...], w1, preferred_element_type=jnp.float32)
    acc = acc + jnp.dot(s2_ref[...], 2.0 * w2, preferred_element_type=jnp.float32)
    o_ref[...] = acc + b_ref[...]


def _linear(x0, x1, s2, w, b):
    m = B * V
    grid = (m // M_TILE,)
    return pl.pallas_call(
        _linear_body,
        grid=grid,
        in_specs=[
            pl.BlockSpec((M_TILE, C), lambda i: (i, 0)),
            pl.BlockSpec((M_TILE, C), lambda i: (i, 0)),
            pl.BlockSpec((M_TILE, C), lambda i: (i, 0)),
            pl.BlockSpec((K, C, OUT), lambda i: (0, 0, 0)),
            pl.BlockSpec((1, OUT), lambda i: (0, 0)),
        ],
        out_specs=pl.BlockSpec((M_TILE, OUT), lambda i: (i, 0)),
        out_shape=jax.ShapeDtypeStruct((m, OUT), jnp.float32),
    )(x0, x1, s2, w, b)


def kernel(input, lap_rows, lap_cols, lap_vals, W, b):
    x0 = jnp.transpose(input, (0, 2, 1)).reshape(B * V, C)

    pad = EPAD - E
    r = jnp.concatenate([lap_rows, jnp.zeros((pad,), jnp.int32)])
    c = jnp.concatenate([lap_cols, jnp.zeros((pad,), jnp.int32)])
    v = jnp.concatenate([lap_vals, jnp.zeros((pad,), jnp.float32)])
    edges = jnp.stack(
        [r.reshape(NS, NCH, CH), c.reshape(NS, NCH, CH)], axis=2)
    evals = v.reshape(NS, NCH, CH)

    y1 = _spmm_sc(edges, evals, x0)
    s2 = _spmm_sc(edges, evals, y1.reshape(B * V, C))

    wp = jnp.transpose(W.reshape(C, K, OUT), (1, 0, 2))
    out = _linear(x0, y1.reshape(B * V, C), s2.reshape(B * V, C),
                  wp, b.reshape(1, OUT))
    return jnp.transpose(out.reshape(B, V, OUT), (0, 2, 1))


# speedup vs baseline: 4.9285x; 1.6716x over previous
import functools

import jax
import jax.numpy as jnp
from jax import lax
from jax.experimental import pallas as pl
from jax.experimental.pallas import tpu as pltpu
from jax.experimental.pallas import tpu_sc as plsc

V = 10000
E = 320000
B = 8
C = 128
K = 3
OUT = 128

NS = 16
NC = 2
CH = 112
NCH = 180
EPAD = NS * NCH * CH
RPS = 624
ZB = 16
TAIL = V - NS * RPS
BPC = B // NC
NEB = 4
NGB = 3

M_TILE = 1000


def _spmm_sc(edges, evals, x_flat):
    mesh = plsc.VectorSubcoreMesh(core_axis_name="c", subcore_axis_name="s")

    @functools.partial(
        pl.kernel,
        mesh=mesh,
        out_type=jax.ShapeDtypeStruct((B, V, C), jnp.float32),
        scratch_types=[
            pltpu.VMEM((NEB, 2, CH), jnp.int32),
            pltpu.VMEM((NEB, CH), jnp.float32),
            pltpu.VMEM((2, CH), jnp.int32),
            pltpu.VMEM((NGB, CH, C), jnp.float32),
            pltpu.VMEM((ZB, C), jnp.float32),
            pltpu.VMEM_SHARED((V, C), jnp.float32),
            pltpu.SemaphoreType.DMA((NEB,)),
            pltpu.SemaphoreType.DMA((NGB,)),
            pltpu.SemaphoreType.DMA((2,)),
        ],
    )
    def k(edges_h, evals_h, x_h, y_h, ebuf, vbuf, idxb, gbuf, zbuf, acc,
          esems, gsems, ssems):
        cid = lax.axis_index("c")
        sid = lax.axis_index("s")

        zero16 = jnp.zeros((16,), jnp.float32)
        for r in range(ZB):
            for j in range(C // 16):
                zbuf[r, pl.ds(16 * j, 16)] = zero16

        def fire_edges(t, es):
            pltpu.make_async_copy(
                edges_h.at[sid, t], ebuf.at[es], esems.at[es]).start()
            pltpu.make_async_copy(
                evals_h.at[sid, t], vbuf.at[es], esems.at[es]).start()

        def wait_edges(t, es):
            pltpu.make_async_copy(
                edges_h.at[sid, t], ebuf.at[es], esems.at[es]).wait()
            pltpu.make_async_copy(
                evals_h.at[sid, t], vbuf.at[es], esems.at[es]).wait()

        for bi in range(BPC):
            b = cid * BPC + bi
            boff = jnp.full((16,), b * V, jnp.int32)

            for i in range(RPS // ZB):
                pltpu.sync_copy(zbuf, acc.at[pl.ds(sid * RPS + i * ZB, ZB)])

            @pl.when(sid == NS - 1)
            def _():
                pltpu.sync_copy(zbuf.at[pl.ds(0, TAIL)],
                                acc.at[pl.ds(NS * RPS, TAIL)])

            plsc.subcore_barrier()

            def stage_and_fire_gather(t, es, islot, gs):
                for g in range(CH // 16):
                    sl = pl.ds(16 * g, 16)
                    idxb[islot, sl] = ebuf[es, 1, sl] + boff
                pltpu.make_async_copy(
                    x_h.at[idxb.at[islot]], gbuf.at[gs], gsems.at[gs]
                ).start()

            def scatter_desc(gs, es, sslot):
                return pltpu.make_async_copy(
                    gbuf.at[gs], acc.at[ebuf.at[es, 0]], ssems.at[sslot])

            fire_edges(0, 0)
            fire_edges(1, 1)
            wait_edges(0, 0)
            stage_and_fire_gather(0, 0, 0, 0)

            def chunk(t, carry):
                gs = lax.rem(t, NGB)
                es = lax.rem(t, NEB)

                @pl.when(t >= 2)
                def _():
                    scatter_desc(lax.rem(t - 2, NGB), lax.rem(t - 2, NEB),
                                 lax.rem(t - 2, 2)).wait()

                @pl.when(t + 2 < NCH)
                def _():
                    fire_edges(t + 2, lax.rem(t + 2, NEB))

                @pl.when(t + 1 < NCH)
                def _():
                    wait_edges(t + 1, lax.rem(t + 1, NEB))
                    stage_and_fire_gather(t + 1, lax.rem(t + 1, NEB),
                                          lax.rem(t + 1, 2),
                                          lax.rem(t + 1, NGB))

                pltpu.make_async_copy(
                    x_h.at[idxb.at[lax.rem(t, 2)]], gbuf.at[gs], gsems.at[gs]
                ).wait()

                @plsc.parallel_loop(0, CH // 16, unroll=2)
                def grp(g):
                    vv = vbuf[es, pl.ds(16 * g, 16)]
                    e0 = 16 * g
                    for l in range(16):
                        v = vv[l]
                        for j in range(C // 16):
                            sl = pl.ds(16 * j, 16)
                            gbuf[gs, e0 + l, sl] = gbuf[gs, e0 + l, sl] * v
                scatter_desc(gs, es, lax.rem(t, 2)).start(add=True)
                return carry

            lax.fori_loop(0, NCH, chunk, 0)
            scatter_desc(lax.rem(NCH - 2, NGB), lax.rem(NCH - 2, NEB),
                         lax.rem(NCH - 2, 2)).wait()
            scatter_desc(lax.rem(NCH - 1, NGB), lax.rem(NCH - 1, NEB),
                         lax.rem(NCH - 1, 2)).wait()
            plsc.subcore_barrier()

            for i in range(RPS // ZB):
                r0 = sid * RPS + i * ZB
                pltpu.sync_copy(acc.at[pl.ds(r0, ZB)],
                                y_h.at[b, pl.ds(r0, ZB)])

            @pl.when(sid == NS - 1)
            def _():
                r0 = NS * RPS
                pltpu.sync_copy(acc.at[pl.ds(r0, TAIL)],
                                y_h.at[b, pl.ds(r0, TAIL)])

    return k(edges, evals, x_flat)


def _linear_body(x0_ref, x1_ref, s2_ref, w_ref, b_ref, o_ref):
    w0 = w_ref[0]
    w1 = w_ref[1]
    w2 = w_ref[2]
    acc = jnp.dot(x0_ref[...], w0 - w2, preferred_element_type=jnp.float32)
    acc = acc + jnp.dot(x1_ref[...], w1, preferred_element_type=jnp.float32)
    acc = acc + jnp.dot(s2_ref[...], 2.0 * w2, preferred_element_type=jnp.float32)
    o_ref[...] = acc + b_ref[...]


def _linear(x0, x1, s2, w, b):
    m = B * V
    grid = (m // M_TILE,)
    return pl.pallas_call(
        _linear_body,
        grid=grid,
        in_specs=[
            pl.BlockSpec((M_TILE, C), lambda i: (i, 0)),
            pl.BlockSpec((M_TILE, C), lambda i: (i, 0)),
            pl.BlockSpec((M_TILE, C), lambda i: (i, 0)),
            pl.BlockSpec((K, C, OUT), lambda i: (0, 0, 0)),
            pl.BlockSpec((1, OUT), lambda i: (0, 0)),
        ],
        out_specs=pl.BlockSpec((M_TILE, OUT), lambda i: (i, 0)),
        out_shape=jax.ShapeDtypeStruct((m, OUT), jnp.float32),
    )(x0, x1, s2, w, b)


def kernel(input, lap_rows, lap_cols, lap_vals, W, b):
    x0 = jnp.transpose(input, (0, 2, 1)).reshape(B * V, C)

    pad = EPAD - E
    r = jnp.concatenate([lap_rows, jnp.zeros((pad,), jnp.int32)])
    c = jnp.concatenate([lap_cols, jnp.zeros((pad,), jnp.int32)])
    v = jnp.concatenate([lap_vals, jnp.zeros((pad,), jnp.float32)])
    edges = jnp.stack(
        [r.reshape(NS, NCH, CH), c.reshape(NS, NCH, CH)], axis=2)
    evals = v.reshape(NS, NCH, CH)

    y1 = _spmm_sc(edges, evals, x0)
    s2 = _spmm_sc(edges, evals, y1.reshape(B * V, C))

    wp = jnp.transpose(W.reshape(C, K, OUT), (1, 0, 2))
    out = _linear(x0, y1.reshape(B * V, C), s2.reshape(B * V, C),
                  wp, b.reshape(1, OUT))
    return jnp.transpose(out.reshape(B, V, OUT), (0, 2, 1))
